# baseline (speedup 1.0000x reference)
import functools
import math

import jax
import jax.numpy as jnp
from jax.experimental import pallas as pl
from jax.experimental.pallas import tpu as pltpu

N_HEADS = 16
_FACTOR = 5
_N_ITERS = 7


def _proj_kernel(x_ref, w_ref, b_ref, o_ref):
    acc = jax.lax.dot_general(x_ref[...], w_ref[...], (((1,), (1,)), ((), ())),
                              preferred_element_type=jnp.float32)
    o_ref[...] = acc + b_ref[0]


def _project(x, W, b, c_blk):
    N, D = x.shape
    b3 = b.reshape(D // c_blk, 1, c_blk)
    return pl.pallas_call(
        _proj_kernel,
        grid=(D // c_blk,),
        in_specs=[
            pl.BlockSpec((N, D), lambda c: (0, 0)),
            pl.BlockSpec((c_blk, D), lambda c: (c, 0)),
            pl.BlockSpec((1, 1, c_blk), lambda c: (c, 0, 0)),
        ],
        out_specs=pl.BlockSpec((N, c_blk), lambda c: (0, c)),
        out_shape=jax.ShapeDtypeStruct((N, D), jnp.float32),
    )(x, W, b3)


def _attn_kernel(q_ref, keys_ref, values_ref, wk_ref, bk_ref, wv_ref,
                 bv_ref, o_ref, kh_ref, vh_ref, *, U, n_iters):
    @pl.when(pl.program_id(1) == 0)
    def _():
        kh_ref[...] = jax.lax.dot_general(
            keys_ref[...], wk_ref[...], (((1,), (1,)), ((), ())),
            preferred_element_type=jnp.float32) + bk_ref[0]
        vh_ref[...] = jax.lax.dot_general(
            values_ref[...], wv_ref[...], (((1,), (1,)), ((), ())),
            preferred_element_type=jnp.float32) + bv_ref[0]

    q = q_ref[...]
    k = kh_ref[...]
    v = vh_ref[...]
    s = jax.lax.dot_general(q, k, (((1,), (1,)), ((), ())),
                            preferred_element_type=jnp.float32)
    S_ = s.shape[1]
    cm = s[:, 0:128]
    for j in range(1, S_ // 128):
        cm = jnp.maximum(cm, s[:, j * 128:(j + 1) * 128])
    m = jnp.max(cm, axis=-1, keepdims=True)
    cm2 = jnp.maximum(cm[:, 0:64], cm[:, 64:128])
    lo0 = jnp.min(cm2, axis=-1, keepdims=True)
    kcnt = jnp.float32(U)

    def body(_, carry):
        lo, hi = carry
        mid = 0.5 * (lo + hi)
        cnt = jnp.sum((s >= mid).astype(jnp.float32), axis=-1, keepdims=True)
        pred = cnt >= kcnt
        return jnp.where(pred, mid, lo), jnp.where(pred, hi, mid)

    lo, _ = jax.lax.fori_loop(0, n_iters, body, (lo0, m))
    mprime = jnp.maximum(m, 0.0)
    bg = jnp.exp(-mprime)
    p = jnp.where(s >= lo, jnp.exp(s - mprime), bg).astype(jnp.bfloat16)
    v1 = jnp.concatenate(
        [v.astype(jnp.bfloat16),
         jnp.ones((v.shape[0], 128), jnp.bfloat16)], axis=1)
    oz = jax.lax.dot_general(p, v1, (((1,), (0,)), ((), ())),
                             preferred_element_type=jnp.float32)
    o_ref[...] = oz[:, :v.shape[1]] / oz[:, v.shape[1]:v.shape[1] + 1]


def _attention(q, keys, values, Wk, bk, Wv, bv, U, l_blk, H):
    L, D = q.shape
    S = keys.shape[0]
    d = D // H
    bk3 = bk.reshape(H, 1, d)
    bv3 = bv.reshape(H, 1, d)
    return pl.pallas_call(
        functools.partial(_attn_kernel, U=U, n_iters=_N_ITERS),
        grid=(H, L // l_blk),
        in_specs=[
            pl.BlockSpec((l_blk, d), lambda h, l: (l, h)),
            pl.BlockSpec((S, D), lambda h, l: (0, 0)),
            pl.BlockSpec((S, D), lambda h, l: (0, 0)),
            pl.BlockSpec((d, D), lambda h, l: (h, 0)),
            pl.BlockSpec((1, 1, d), lambda h, l: (h, 0, 0)),
            pl.BlockSpec((d, D), lambda h, l: (h, 0)),
            pl.BlockSpec((1, 1, d), lambda h, l: (h, 0, 0)),
        ],
        out_specs=pl.BlockSpec((l_blk, d), lambda h, l: (l, h)),
        out_shape=jax.ShapeDtypeStruct((L, D), jnp.float32),
        scratch_shapes=[
            pltpu.VMEM((S, d), jnp.float32),
            pltpu.VMEM((S, d), jnp.float32),
        ],
    )(q, keys, values, Wk, bk3, Wv, bv3)


def kernel(queries, keys, values, Wq, bq, Wk, bk, Wv, bv):
    B_, L, D = queries.shape
    S = keys.shape[1]
    U = int(_FACTOR * math.log(L))
    q = _project(queries.reshape(B_ * L, D), Wq, bq, 128)
    out = _attention(q, keys.reshape(B_ * S, D), values.reshape(B_ * S, D),
                     Wk, bk, Wv, bv, U, min(1024, L), N_HEADS)
    return out.reshape(B_, L, D)


# speedup vs baseline: 1.0002x; 1.0002x over previous
import functools
import math

import jax
import jax.numpy as jnp
from jax.experimental import pallas as pl
from jax.experimental.pallas import tpu as pltpu

N_HEADS = 16
_FACTOR = 5
_N_ITERS = 7


def _proj_kernel(x_ref, w_ref, b_ref, o_ref):
    acc = jax.lax.dot_general(x_ref[...], w_ref[...], (((1,), (1,)), ((), ())),
                              preferred_element_type=jnp.float32)
    o_ref[...] = acc + b_ref[0]


def _project(x, W, b, c_blk):
    N, D = x.shape
    b3 = b.reshape(D // c_blk, 1, c_blk)
    return pl.pallas_call(
        _proj_kernel,
        grid=(D // c_blk,),
        in_specs=[
            pl.BlockSpec((N, D), lambda c: (0, 0)),
            pl.BlockSpec((c_blk, D), lambda c: (c, 0)),
            pl.BlockSpec((1, 1, c_blk), lambda c: (c, 0, 0)),
        ],
        out_specs=pl.BlockSpec((N, c_blk), lambda c: (0, c)),
        out_shape=jax.ShapeDtypeStruct((N, D), jnp.float32),
    )(x, W, b3)


def _attn_kernel(q_ref, keys_ref, values_ref, wk_ref, bk_ref, wv_ref,
                 bv_ref, o_ref, kh_ref, vh_ref, *, U, n_iters):
    @pl.when(pl.program_id(1) == 0)
    def _():
        kh_ref[...] = jax.lax.dot_general(
            keys_ref[...], wk_ref[...], (((1,), (1,)), ((), ())),
            preferred_element_type=jnp.float32) + bk_ref[0]
        vh_ref[...] = jax.lax.dot_general(
            values_ref[...], wv_ref[...], (((1,), (1,)), ((), ())),
            preferred_element_type=jnp.float32) + bv_ref[0]

    q = q_ref[...]
    k = kh_ref[...]
    v = vh_ref[...]
    s = jax.lax.dot_general(q, k, (((1,), (1,)), ((), ())),
                            preferred_element_type=jnp.float32)
    S_ = s.shape[1]
    cm = s[:, 0:128]
    for j in range(1, S_ // 128):
        cm = jnp.maximum(cm, s[:, j * 128:(j + 1) * 128])
    m = jnp.max(cm, axis=-1, keepdims=True)
    cm2 = jnp.maximum(cm, jnp.concatenate([cm[:, 64:], cm[:, :64]], axis=1))
    lo0 = jnp.min(cm2, axis=-1, keepdims=True)
    kcnt = jnp.float32(U)

    def body(_, carry):
        lo, hi = carry
        mid = 0.5 * (lo + hi)
        cnt = jnp.sum((s >= mid).astype(jnp.float32), axis=-1, keepdims=True)
        pred = cnt >= kcnt
        return jnp.where(pred, mid, lo), jnp.where(pred, hi, mid)

    lo, _ = jax.lax.fori_loop(0, n_iters, body, (lo0, m))
    mprime = jnp.maximum(m, 0.0)
    bg = jnp.exp(-mprime)
    p = jnp.where(s >= lo, jnp.exp(s - mprime), bg).astype(jnp.bfloat16)
    v1 = jnp.concatenate(
        [v.astype(jnp.bfloat16),
         jnp.ones((v.shape[0], 128), jnp.bfloat16)], axis=1)
    oz = jax.lax.dot_general(p, v1, (((1,), (0,)), ((), ())),
                             preferred_element_type=jnp.float32)
    o_ref[...] = oz[:, :v.shape[1]] / oz[:, v.shape[1]:v.shape[1] + 1]


def _attention(q, keys, values, Wk, bk, Wv, bv, U, l_blk, H):
    L, D = q.shape
    S = keys.shape[0]
    d = D // H
    bk3 = bk.reshape(H, 1, d)
    bv3 = bv.reshape(H, 1, d)
    return pl.pallas_call(
        functools.partial(_attn_kernel, U=U, n_iters=_N_ITERS),
        grid=(H, L // l_blk),
        in_specs=[
            pl.BlockSpec((l_blk, d), lambda h, l: (l, h)),
            pl.BlockSpec((S, D), lambda h, l: (0, 0)),
            pl.BlockSpec((S, D), lambda h, l: (0, 0)),
            pl.BlockSpec((d, D), lambda h, l: (h, 0)),
            pl.BlockSpec((1, 1, d), lambda h, l: (h, 0, 0)),
            pl.BlockSpec((d, D), lambda h, l: (h, 0)),
            pl.BlockSpec((1, 1, d), lambda h, l: (h, 0, 0)),
        ],
        out_specs=pl.BlockSpec((l_blk, d), lambda h, l: (l, h)),
        out_shape=jax.ShapeDtypeStruct((L, D), jnp.float32),
        scratch_shapes=[
            pltpu.VMEM((S, d), jnp.float32),
            pltpu.VMEM((S, d), jnp.float32),
        ],
    )(q, keys, values, Wk, bk3, Wv, bv3)


def kernel(queries, keys, values, Wq, bq, Wk, bk, Wv, bv):
    B_, L, D = queries.shape
    S = keys.shape[1]
    U = int(_FACTOR * math.log(L))
    q = _project(queries.reshape(B_ * L, D), Wq, bq, 128)
    out = _attention(q, keys.reshape(B_ * S, D), values.reshape(B_ * S, D),
                     Wk, bk, Wv, bv, U, min(1024, L), N_HEADS)
    return out.reshape(B_, L, D)


# speedup vs baseline: 1.0300x; 1.0298x over previous
import functools
import math

import jax
import jax.numpy as jnp
from jax.experimental import pallas as pl
from jax.experimental.pallas import tpu as pltpu

N_HEADS = 16
_FACTOR = 5
_N_ITERS = 8


def _proj_kernel(x_ref, w_ref, b_ref, o_ref):
    acc = jax.lax.dot_general(x_ref[...], w_ref[...], (((1,), (1,)), ((), ())),
                              preferred_element_type=jnp.float32)
    o_ref[...] = acc + b_ref[0]


def _project(x, W, b, c_blk):
    N, D = x.shape
    b3 = b.reshape(D // c_blk, 1, c_blk)
    return pl.pallas_call(
        _proj_kernel,
        grid=(D // c_blk,),
        in_specs=[
            pl.BlockSpec((N, D), lambda c: (0, 0)),
            pl.BlockSpec((c_blk, D), lambda c: (c, 0)),
            pl.BlockSpec((1, 1, c_blk), lambda c: (c, 0, 0)),
        ],
        out_specs=pl.BlockSpec((N, c_blk), lambda c: (0, c)),
        out_shape=jax.ShapeDtypeStruct((N, D), jnp.float32),
    )(x, W, b3)


def _attn_kernel(q_ref, keys_ref, values_ref, wk_ref, bk_ref, wv_ref,
                 bv_ref, o_ref, kh_ref, vh_ref, *, U, n_iters):
    @pl.when(pl.program_id(1) == 0)
    def _():
        kh_ref[...] = jax.lax.dot_general(
            keys_ref[...], wk_ref[...], (((1,), (1,)), ((), ())),
            preferred_element_type=jnp.float32) + bk_ref[0]
        vh_ref[...] = jax.lax.dot_general(
            values_ref[...], wv_ref[...], (((1,), (1,)), ((), ())),
            preferred_element_type=jnp.float32) + bv_ref[0]

    q = q_ref[...]
    k = kh_ref[...]
    v = vh_ref[...]
    s = jax.lax.dot_general(q, k, (((1,), (1,)), ((), ())),
                            preferred_element_type=jnp.float32)
    S_ = s.shape[1]
    cm = s[:, 0:128]
    for j in range(1, S_ // 128):
        cm = jnp.maximum(cm, s[:, j * 128:(j + 1) * 128])
    m = jnp.max(cm, axis=-1, keepdims=True)
    lo0 = jnp.min(cm, axis=-1, keepdims=True)
    kcnt = jnp.float32(U)

    def body(_, carry):
        lo, hi = carry
        mid = 0.5 * (lo + hi)
        cnt = jnp.sum((s >= mid).astype(jnp.float32), axis=-1, keepdims=True)
        pred = cnt >= kcnt
        return jnp.where(pred, mid, lo), jnp.where(pred, hi, mid)

    lo, _ = jax.lax.fori_loop(0, n_iters, body, (lo0, m))
    mprime = jnp.maximum(m, 0.0)
    bg = jnp.exp(-mprime)
    p = jnp.where(s >= lo, jnp.exp(s - mprime), bg).astype(jnp.bfloat16)
    v1 = jnp.concatenate(
        [v.astype(jnp.bfloat16),
         jnp.ones((v.shape[0], 128), jnp.bfloat16)], axis=1)
    oz = jax.lax.dot_general(p, v1, (((1,), (0,)), ((), ())),
                             preferred_element_type=jnp.float32)
    o_ref[...] = oz[:, :v.shape[1]] / oz[:, v.shape[1]:v.shape[1] + 1]


def _attention(q, keys, values, Wk, bk, Wv, bv, U, l_blk, H):
    L, D = q.shape
    S = keys.shape[0]
    d = D // H
    bk3 = bk.reshape(H, 1, d)
    bv3 = bv.reshape(H, 1, d)
    return pl.pallas_call(
        functools.partial(_attn_kernel, U=U, n_iters=_N_ITERS),
        grid=(H, L // l_blk),
        in_specs=[
            pl.BlockSpec((l_blk, d), lambda h, l: (l, h)),
            pl.BlockSpec((S, D), lambda h, l: (0, 0)),
            pl.BlockSpec((S, D), lambda h, l: (0, 0)),
            pl.BlockSpec((d, D), lambda h, l: (h, 0)),
            pl.BlockSpec((1, 1, d), lambda h, l: (h, 0, 0)),
            pl.BlockSpec((d, D), lambda h, l: (h, 0)),
            pl.BlockSpec((1, 1, d), lambda h, l: (h, 0, 0)),
        ],
        out_specs=pl.BlockSpec((l_blk, d), lambda h, l: (l, h)),
        out_shape=jax.ShapeDtypeStruct((L, D), jnp.float32),
        scratch_shapes=[
            pltpu.VMEM((S, d), jnp.float32),
            pltpu.VMEM((S, d), jnp.float32),
        ],
    )(q, keys, values, Wk, bk3, Wv, bv3)


def kernel(queries, keys, values, Wq, bq, Wk, bk, Wv, bv):
    B_, L, D = queries.shape
    S = keys.shape[1]
    U = int(_FACTOR * math.log(L))
    q = _project(queries.reshape(B_ * L, D), Wq, bq, 128)
    out = _attention(q, keys.reshape(B_ * S, D), values.reshape(B_ * S, D),
                     Wk, bk, Wv, bv, U, min(1024, L), N_HEADS)
    return out.reshape(B_, L, D)
